# baseline (speedup 1.0000x reference)
import jax
import jax.numpy as jnp
from jax import lax
from jax.experimental import pallas as pl
from jax.experimental.pallas import tpu as pltpu
from jax.experimental.pallas import tpu_sc as plsc

B = 16384
F = 26
D = 128
V = 26000
NC = 2
NS = 16
NW = NC * NS
RPW = B // NW
CR = 4
NCHUNK = RPW // CR
IPC = CR * F
NBUF = 4
LANES = 16
DCH = D // LANES


def _fm_body(xi_hbm, fl_hbm, bias_hbm, fe_hbm, out_hbm,
             xi_v, fl_v, bias_v, eb0, eb1, eb2, eb3, zbuf, ybuf, obuf,
             sm0, sm1, sm2, sm3):
    ebufs = (eb0, eb1, eb2, eb3)
    sems = (sm0, sm1, sm2, sm3)
    wid = lax.axis_index("s") * NC + lax.axis_index("c")
    base = wid * RPW
    ibase = base * F

    pltpu.sync_copy(xi_hbm.at[pl.ds(ibase, RPW * F)], xi_v)
    pltpu.sync_copy(fl_hbm, fl_v)
    pltpu.sync_copy(bias_hbm, bias_v)

    def fire(chunk, ebuf, sem):
        idx = xi_v.at[pl.ds(chunk * IPC, IPC)]
        pltpu.async_copy(fe_hbm.at[idx], ebuf, sem)

    def drain(ebuf, sem):
        idx = xi_v.at[pl.ds(0, IPC)]
        pltpu.make_async_copy(fe_hbm.at[idx], ebuf, sem).wait()

    for b in range(NBUF):
        fire(b, ebufs[b], sems[b])

    iota = lax.iota(jnp.int32, LANES)
    stride = iota * F

    def fl_group(g, _):
        def fl_field(f, acc):
            addr = stride + (g * (LANES * F) + f)
            xiv = plsc.load_gather(xi_v, [addr])
            return acc + plsc.load_gather(fl_v, [xiv])

        flacc = lax.fori_loop(0, F, fl_field,
                              jnp.zeros((LANES,), jnp.float32))
        zbuf[pl.ds(g * LANES, LANES)] = flacc
        return 0

    lax.fori_loop(0, RPW // LANES, fl_group, 0)

    lane_is_last = iota == (LANES - 1)
    zero = jnp.zeros((LANES,), jnp.float32)

    def row_body(r, carry, chunk, ebuf):
        del carry
        s = [zero] * DCH
        q = zero
        for f in range(F):
            row = r * F + f
            for c in range(DCH):
                v = ebuf[row, pl.ds(c * LANES, LANES)]
                s[c] = s[c] + v
                q = q + v * v
        s2 = s[0] * s[0]
        for c in range(1, DCH):
            s2 = s2 + s[c] * s[c]
        rowv = s2 - q
        rowtot = plsc.cumsum(0.5 * rowv)
        idxv = jnp.full((LANES,), chunk * CR + r, jnp.int32)
        plsc.store_scatter(ybuf, [idxv], rowtot, mask=lane_is_last)
        return 0

    def quad_body(p, _):
        for b in range(NBUF):
            chunk = NBUF * p + b
            ebuf = ebufs[b]
            sem = sems[b]
            drain(ebuf, sem)
            lax.fori_loop(
                0, CR,
                lambda r, cy, chunk=chunk, ebuf=ebuf:
                    row_body(r, cy, chunk, ebuf),
                0)

            @pl.when(p < NCHUNK // NBUF - 1)
            def _():
                fire(chunk + NBUF, ebuf, sem)
        return 0

    lax.fori_loop(0, NCHUNK // NBUF, quad_body, 0)

    biasv = bias_v[...]

    def out_group(g, _):
        zv = zbuf[pl.ds(g * LANES, LANES)] + ybuf[pl.ds(g * LANES, LANES)]
        zv = zv + biasv
        obuf[pl.ds(g * LANES, LANES)] = 1.0 / (1.0 + jnp.exp(-zv))
        return 0

    lax.fori_loop(0, RPW // LANES, out_group, 0)
    pltpu.sync_copy(obuf, out_hbm.at[pl.ds(base, RPW)])


@jax.jit
def _fm_sc(xi_flat, fl, bias16, fe):
    mesh = plsc.VectorSubcoreMesh(core_axis_name="c", subcore_axis_name="s")
    run = pl.kernel(
        _fm_body,
        mesh=mesh,
        compiler_params=pltpu.CompilerParams(needs_layout_passes=False),
        out_type=jax.ShapeDtypeStruct((B,), jnp.float32),
        scratch_types=[
            pltpu.VMEM((RPW * F,), jnp.int32),
            pltpu.VMEM((V,), jnp.float32),
            pltpu.VMEM((LANES,), jnp.float32),
            pltpu.VMEM((IPC, D), jnp.float32),
            pltpu.VMEM((IPC, D), jnp.float32),
            pltpu.VMEM((IPC, D), jnp.float32),
            pltpu.VMEM((IPC, D), jnp.float32),
            pltpu.VMEM((RPW,), jnp.float32),
            pltpu.VMEM((RPW,), jnp.float32),
            pltpu.VMEM((RPW,), jnp.float32),
            pltpu.SemaphoreType.DMA,
            pltpu.SemaphoreType.DMA,
            pltpu.SemaphoreType.DMA,
            pltpu.SemaphoreType.DMA,
        ],
    )
    return run(xi_flat, fl, bias16, fe)


def kernel(x, FL_weight, FL_bias, FE_weight, offsets):
    xi_flat = (x + offsets[None, :]).reshape(-1).astype(jnp.int32)
    fl = FL_weight[:, 0]
    bias16 = jnp.broadcast_to(FL_bias.astype(jnp.float32), (LANES,))
    return _fm_sc(xi_flat, fl, bias16, FE_weight)
